# baseline (speedup 1.0000x reference)
import jax
import jax.numpy as jnp
from jax import lax
from jax.experimental import pallas as pl
from jax.experimental.pallas import tpu as pltpu
from jax.experimental.pallas import tpu_sc as plsc

B, C, H, W = 4, 96, 224, 224
R = B * C
RB = 32
GRID = R // RB
CB_COUNT = C // RB
QCNT = C // 4 + 1
L = 16


def _sc_body(b1_hbm, b2_hbm, cnt_hbm, bn_v, cnt_v, sem):
    wid = lax.axis_index("c") * 16 + lax.axis_index("s")

    @pl.when(wid == 0)
    def _():
        pltpu.sync_copy(b1_hbm, bn_v.at[0, pl.ds(0, C)])
        pltpu.sync_copy(b2_hbm, bn_v.at[1, pl.ds(0, C)])
        for k in range(2):
            for iv in range(C // L):
                sl = pl.ds(iv * L, L)
                bn_v[k, sl] = jnp.abs(bn_v[k, sl])

        def jstep(j, carry):
            out = []
            for k in range(2):
                sj = bn_v[k, pl.ds(j, L)][0]
                for iv in range(C // L):
                    av = bn_v[k, pl.ds(iv * L, L)]
                    out.append(carry[k * (C // L) + iv] +
                               jnp.where(sj <= av, jnp.int32(1),
                                         jnp.int32(0)))
            return tuple(out)

        init = tuple(jnp.zeros((L,), jnp.int32) for _ in range(2 * (C // L)))
        cnt = lax.fori_loop(0, C, jstep, init)
        for k in range(2):
            for iv in range(C // L):
                cnt_v[pl.ds(k * C + iv * L, L)] = cnt[k * (C // L) + iv]
        pltpu.sync_copy(cnt_v.at[pl.ds(0, 2 * C)], cnt_hbm)


def _sc_counts(b1, b2):
    mesh = plsc.VectorSubcoreMesh(core_axis_name="c", subcore_axis_name="s")
    f = pl.kernel(
        _sc_body,
        out_type=jax.ShapeDtypeStruct((2 * C,), jnp.int32),
        mesh=mesh,
        scratch_types=[
            pltpu.VMEM((2, C + L), jnp.float32),
            pltpu.VMEM((2 * C + L,), jnp.int32),
            pltpu.SemaphoreType.DMA,
        ],
        compiler_params=pltpu.CompilerParams(use_tc_tiling_on_sc=True),
    )
    return f(b1, b2)


def _tc_body(x0_ref, x1_ref, c1_ref, c2_ref, o1_ref, o2_ref):
    m1 = jnp.reshape(c1_ref[...] >= QCNT, (RB, 1, 1))
    m2 = jnp.reshape(c2_ref[...] >= QCNT, (RB, 1, 1))
    x0 = x0_ref[...]
    x1 = x1_ref[...]
    o1_ref[...] = jnp.where(m1, x0, x1)
    o2_ref[...] = jnp.where(m2, x1, x0)


def _tc_exchange(x0r, x1r, cnt2d):
    return pl.pallas_call(
        _tc_body,
        grid=(GRID,),
        in_specs=[
            pl.BlockSpec((RB, H, W), lambda i: (i, 0, 0)),
            pl.BlockSpec((RB, H, W), lambda i: (i, 0, 0)),
            pl.BlockSpec((RB, 1), lambda i: (i % CB_COUNT, 0)),
            pl.BlockSpec((RB, 1), lambda i: (CB_COUNT + i % CB_COUNT, 0)),
        ],
        out_specs=[
            pl.BlockSpec((RB, H, W), lambda i: (i, 0, 0)),
            pl.BlockSpec((RB, H, W), lambda i: (i, 0, 0)),
        ],
        out_shape=[
            jax.ShapeDtypeStruct((R, H, W), jnp.float32),
            jax.ShapeDtypeStruct((R, H, W), jnp.float32),
        ],
        compiler_params=pltpu.CompilerParams(
            vmem_limit_bytes=64 * 1024 * 1024),
    )(x0r, x1r, cnt2d, cnt2d)


def kernel(x0, x1, bn1_weight, bn2_weight, bn_threshold):
    del bn_threshold
    x0r = x0.reshape(R, H, W)
    x1r = x1.reshape(R, H, W)
    cnt2d = _sc_counts(bn1_weight, bn2_weight).reshape(2 * C, 1)
    out1, out2 = _tc_exchange(x0r, x1r, cnt2d)
    return (out1.reshape(B, C, H, W), out2.reshape(B, C, H, W))
